# baseline (speedup 1.0000x reference)
import functools

import jax
import jax.numpy as jnp
import numpy as np
from jax import lax
from jax.experimental import pallas as pl
from jax.experimental.pallas import tpu as pltpu
from jax.experimental.pallas import tpu_sc as plsc

_VOCAB = 100000
_EMBED = 64
_NUM_CLASS = 4
_BATCH = 4096
_CUTLEN = 200

_NC = 2
_NS = 16
_NW = _NC * _NS
_BPW = _BATCH // _NW
_TPW = _BPW * _CUTLEN
_G = 128
_NGROUPS = _TPW // _G

_NBUF = 8
_AHEAD = 6

_SEG_TABLE = (
    np.arange(_NS, dtype=np.int32)[:, None] * _BPW
    + (np.arange(_TPW, dtype=np.int32) // _CUTLEN)[None, :]
).reshape(_NS, _NGROUPS, _G)


def _sc_body(text_ref, table_ref, seg_ref, out_ref, idx_v, seg_v, rows_v,
             acc_sh, sem_g, sem_s):
  c = lax.axis_index("c")
  s = lax.axis_index("s")
  wid = c * _NS + s

  pltpu.sync_copy(text_ref.at[pl.ds(wid * _TPW, _TPW)], idx_v)
  pltpu.sync_copy(seg_ref.at[s], seg_v)

  def _zero_row(r, carry):
    for q in range(_EMBED // 16):
      rows_v[0, r, pl.ds(q * 16, 16)] = jnp.zeros((16,), jnp.float32)
    return carry

  lax.fori_loop(0, _BPW, _zero_row, 0)
  pltpu.sync_copy(rows_v.at[0], acc_sh.at[pl.ds(s * _BPW, _BPW)])

  for j in range(_AHEAD):
    pltpu.async_copy(table_ref.at[idx_v.at[pl.ds(j * _G, _G)]],
                     rows_v.at[j], sem_g)

  def _group(j, carry):
    slot = lax.rem(j, _NBUF)
    pltpu.make_async_copy(table_ref.at[idx_v.at[pl.ds(j * _G, _G)]],
                          rows_v.at[slot], sem_g).wait()
    pltpu.async_copy(rows_v.at[slot], acc_sh.at[seg_v.at[j]], sem_s,
                     add=True)

    @pl.when(j >= 1)
    def _drain_prev():
      pslot = lax.rem(j - 1, _NBUF)
      pltpu.make_async_copy(rows_v.at[pslot], acc_sh.at[seg_v.at[j - 1]],
                            sem_s).wait()

    @pl.when(j + _AHEAD < _NGROUPS)
    def _fire_next():
      nslot = lax.rem(j + _AHEAD, _NBUF)
      pltpu.async_copy(table_ref.at[idx_v.at[pl.ds((j + _AHEAD) * _G, _G)]],
                       rows_v.at[nslot], sem_g)

    return carry

  lax.fori_loop(0, _NGROUPS, _group, 0)

  last = _NGROUPS - 1
  pltpu.make_async_copy(rows_v.at[last % _NBUF], acc_sh.at[seg_v.at[last]],
                        sem_s).wait()

  pltpu.sync_copy(acc_sh.at[pl.ds(s * _BPW, _BPW)],
                  out_ref.at[pl.ds(wid * _BPW, _BPW)])


@jax.jit
def _segment_sums(text, table, seg):
  mesh = plsc.VectorSubcoreMesh(core_axis_name="c", subcore_axis_name="s",
                                num_cores=_NC, num_subcores=_NS)
  fn = pl.kernel(
      _sc_body,
      out_type=jax.ShapeDtypeStruct((_BATCH, _EMBED), jnp.float32),
      mesh=mesh,
      scratch_types=[
          pltpu.VMEM((_TPW,), jnp.int32),
          pltpu.VMEM((_NGROUPS, _G), jnp.int32),
          pltpu.VMEM((_NBUF, _BPW, _EMBED), jnp.float32),
          pltpu.VMEM_SHARED((_NS * _BPW, _EMBED), jnp.float32),
          pltpu.SemaphoreType.DMA,
          pltpu.SemaphoreType.DMA,
      ],
      compiler_params=pltpu.CompilerParams(use_tc_tiling_on_sc=False),
  )
  return fn(text, table, seg)


def _tc_body(p_ref, w_ref, b_ref, o_ref):
  logits = jnp.dot(p_ref[...], w_ref[...],
                   preferred_element_type=jnp.float32) + b_ref[...]
  m = jnp.max(logits, axis=1, keepdims=True)
  e = jnp.exp(logits - m)
  o_ref[...] = e / jnp.sum(e, axis=1, keepdims=True)


@jax.jit
def _dense_softmax(pooled_sum, wt, b2):
  return pl.pallas_call(
      _tc_body,
      out_shape=jax.ShapeDtypeStruct((_BATCH, _NUM_CLASS), jnp.float32),
  )(pooled_sum, wt, b2)


def kernel(text, table, W, b):
  wt = (W.astype(jnp.float32) * (1.0 / _CUTLEN)).T
  b2 = b.reshape(1, _NUM_CLASS).astype(jnp.float32)
  seg = jnp.asarray(_SEG_TABLE)

  pooled_sum = _segment_sums(text, table, seg)
  return _dense_softmax(pooled_sum, wt, b2)


# speedup vs baseline: 1.0094x; 1.0094x over previous
import functools

import jax
import jax.numpy as jnp
import numpy as np
from jax import lax
from jax.experimental import pallas as pl
from jax.experimental.pallas import tpu as pltpu
from jax.experimental.pallas import tpu_sc as plsc

_VOCAB = 100000
_EMBED = 64
_NUM_CLASS = 4
_BATCH = 4096
_CUTLEN = 200

_NC = 2
_NS = 16
_NW = _NC * _NS
_BPW = _BATCH // _NW
_TPW = _BPW * _CUTLEN
_G = 128
_NGROUPS = _TPW // _G

_NBUF = 8
_AHEAD = 6

_DIV200_MUL = 5243
_DIV200_SHIFT = 20


def _sc_body(text_ref, table_ref, out_ref, idx_v, seg_v, rows_v,
             acc_sh, sem_g, sem_s):
  c = lax.axis_index("c")
  s = lax.axis_index("s")
  wid = c * _NS + s

  pltpu.sync_copy(text_ref.at[pl.ds(wid * _TPW, _TPW)], idx_v)

  def _zero_row(r, carry):
    for q in range(_EMBED // 16):
      rows_v[0, r, pl.ds(q * 16, 16)] = jnp.zeros((16,), jnp.float32)
    return carry

  lax.fori_loop(0, _BPW, _zero_row, 0)
  pltpu.sync_copy(rows_v.at[0], acc_sh.at[pl.ds(s * _BPW, _BPW)])

  lanes = lax.iota(jnp.int32, 16)
  base = s * _BPW

  def _gen_seg(j, carry):
    for q in range(_G // 16):
      p = lanes + (j * _G + q * 16)
      seg_v[j, pl.ds(q * 16, 16)] = (
          base + ((p * _DIV200_MUL) >> _DIV200_SHIFT))
    return carry

  lax.fori_loop(0, _NGROUPS, _gen_seg, 0)

  for j in range(_AHEAD):
    pltpu.async_copy(table_ref.at[idx_v.at[pl.ds(j * _G, _G)]],
                     rows_v.at[j], sem_g)

  def _group(j, carry):
    slot = lax.rem(j, _NBUF)
    pltpu.make_async_copy(table_ref.at[idx_v.at[pl.ds(j * _G, _G)]],
                          rows_v.at[slot], sem_g).wait()
    pltpu.async_copy(rows_v.at[slot], acc_sh.at[seg_v.at[j]], sem_s,
                     add=True)

    @pl.when(j >= 1)
    def _drain_prev():
      pslot = lax.rem(j - 1, _NBUF)
      pltpu.make_async_copy(rows_v.at[pslot], acc_sh.at[seg_v.at[j - 1]],
                            sem_s).wait()

    @pl.when(j + _AHEAD < _NGROUPS)
    def _fire_next():
      nslot = lax.rem(j + _AHEAD, _NBUF)
      pltpu.async_copy(table_ref.at[idx_v.at[pl.ds((j + _AHEAD) * _G, _G)]],
                       rows_v.at[nslot], sem_g)

    return carry

  lax.fori_loop(0, _NGROUPS, _group, 0)

  last = _NGROUPS - 1
  pltpu.make_async_copy(rows_v.at[last % _NBUF], acc_sh.at[seg_v.at[last]],
                        sem_s).wait()

  pltpu.sync_copy(acc_sh.at[pl.ds(s * _BPW, _BPW)],
                  out_ref.at[pl.ds(wid * _BPW, _BPW)])


@jax.jit
def _segment_sums(text, table):
  mesh = plsc.VectorSubcoreMesh(core_axis_name="c", subcore_axis_name="s",
                                num_cores=_NC, num_subcores=_NS)
  fn = pl.kernel(
      _sc_body,
      out_type=jax.ShapeDtypeStruct((_BATCH, _EMBED), jnp.float32),
      mesh=mesh,
      scratch_types=[
          pltpu.VMEM((_TPW,), jnp.int32),
          pltpu.VMEM((_NGROUPS, _G), jnp.int32),
          pltpu.VMEM((_NBUF, _BPW, _EMBED), jnp.float32),
          pltpu.VMEM_SHARED((_NS * _BPW, _EMBED), jnp.float32),
          pltpu.SemaphoreType.DMA,
          pltpu.SemaphoreType.DMA,
      ],
      compiler_params=pltpu.CompilerParams(use_tc_tiling_on_sc=False),
  )
  return fn(text, table)


def _tc_body(p_ref, w_ref, b_ref, o_ref):
  logits = jnp.dot(p_ref[...], w_ref[...],
                   preferred_element_type=jnp.float32) + b_ref[...]
  m = jnp.max(logits, axis=1, keepdims=True)
  e = jnp.exp(logits - m)
  o_ref[...] = e / jnp.sum(e, axis=1, keepdims=True)


@jax.jit
def _dense_softmax(pooled_sum, wt, b2):
  return pl.pallas_call(
      _tc_body,
      out_shape=jax.ShapeDtypeStruct((_BATCH, _NUM_CLASS), jnp.float32),
  )(pooled_sum, wt, b2)


def kernel(text, table, W, b):
  wt = (W.astype(jnp.float32) * (1.0 / _CUTLEN)).T
  b2 = b.reshape(1, _NUM_CLASS).astype(jnp.float32)

  pooled_sum = _segment_sums(text, table)
  return _dense_softmax(pooled_sum, wt, b2)
